# baseline (speedup 1.0000x reference)
import jax
import jax.numpy as jnp
from jax import lax
from jax.experimental import pallas as pl
from jax.experimental.pallas import tpu as pltpu
from jax.experimental.pallas import tpu_sc as plsc

TABLE_SIZE = 262144
MASK = TABLE_SIZE - 1
PRIMES = (6700417, 15485863, 32452843, 49979687)
NUM_HEADS = 4
D = 128
L = 16
NC, NS = 2, 16
N_KEYS = 4096 * 64
KEYS_PER_W = N_KEYS // (NC * NS)
CHUNK = 128
N_CHUNKS = KEYS_PER_W // CHUNK
NBUF = 4
N_QUADS = N_CHUNKS // NBUF


def _sc_body(keys_hbm, table_hbm, out_hbm, keys_v, idx_v, rows_v, sems):
    si = lax.axis_index("s")
    ci = lax.axis_index("c")
    gsem = [sems.at[jnp.int32(b)] for b in range(NBUF)]
    wsem = [sems.at[jnp.int32(NBUF + b)] for b in range(NBUF)]

    pltpu.sync_copy(keys_hbm.at[si, ci], keys_v)

    zvec = jnp.zeros((L,), jnp.float32)

    def _zero(buf):
        def _z(i, carry):
            for cc in range(D // L):
                rows_v[buf, i, pl.ds(cc * L, L)] = zvec
            return carry

        lax.fori_loop(jnp.int32(0), jnp.int32(CHUNK), _z, 0)

    def _hash_and_issue(g, buf):
        off = g * jnp.int32(CHUNK)
        for v in range(CHUNK // L):
            k = keys_v[pl.ds(off + jnp.int32(v * L), L)]
            sl = pl.ds(v * L, L)
            for j in range(NUM_HEADS):
                idx_v[buf, j, sl] = (k * jnp.int32(PRIMES[j])) & jnp.int32(MASK)
        for j in range(NUM_HEADS):
            pltpu.async_copy(
                table_hbm.at[idx_v.at[jnp.int32(buf), jnp.int32(j)]],
                rows_v.at[jnp.int32(buf)],
                gsem[buf],
                add=True,
            )

    def _wait_gathers(buf):
        for j in range(NUM_HEADS):
            pltpu.make_async_copy(
                table_hbm.at[idx_v.at[jnp.int32(buf), jnp.int32(j)]],
                rows_v.at[jnp.int32(buf)],
                gsem[buf],
            ).wait()

    def _scale(buf):
        def _key(i, carry):
            for cc in range(D // L):
                sl = pl.ds(cc * L, L)
                rows_v[buf, i, sl] = rows_v[buf, i, sl] * jnp.float32(0.25)
            return carry

        lax.fori_loop(jnp.int32(0), jnp.int32(CHUNK), _key, 0)

    def _wait_write(buf):
        pltpu.make_async_copy(
            rows_v.at[jnp.int32(buf)], out_hbm.at[si, ci, jnp.int32(buf)], wsem[buf]
        ).wait()

    def _start_write(g, buf):
        pltpu.async_copy(
            rows_v.at[jnp.int32(buf)], out_hbm.at[si, ci, g], wsem[buf]
        )

    for b in range(NBUF):
        _zero(b)
    _hash_and_issue(jnp.int32(0), 0)
    _hash_and_issue(jnp.int32(1), 1)

    def _quad(q, carry):
        for k in range(NBUF):
            g = q * jnp.int32(NBUF) + jnp.int32(k)
            nb = (k + 2) % NBUF
            gn = g + jnp.int32(2)

            _wait_gathers(k)
            _scale(k)
            _start_write(g, k)

            @pl.when(jnp.logical_and(gn >= jnp.int32(NBUF), gn < jnp.int32(N_CHUNKS)))
            def _():
                _wait_write(nb)
                _zero(nb)

            @pl.when(gn < jnp.int32(N_CHUNKS))
            def _():
                _hash_and_issue(gn, nb)

        return carry

    lax.fori_loop(jnp.int32(0), jnp.int32(N_QUADS), _quad, 0)
    for b in range(NBUF):
        _wait_write(b)


@jax.jit
def _sc_lookup(keys_grouped, table):
    mesh = plsc.VectorSubcoreMesh(
        core_axis_name="c", subcore_axis_name="s", num_cores=NC, num_subcores=NS
    )
    f = pl.kernel(
        _sc_body,
        out_type=jax.ShapeDtypeStruct((NS, NC, N_CHUNKS, CHUNK, D), jnp.float32),
        mesh=mesh,
        scratch_types=[
            pltpu.VMEM((KEYS_PER_W,), jnp.int32),
            pltpu.VMEM((NBUF, NUM_HEADS, CHUNK), jnp.int32),
            pltpu.VMEM((NBUF, CHUNK, D), jnp.float32),
            pltpu.SemaphoreType.DMA((2 * NBUF,)),
        ],
    )
    return f(keys_grouped, table)


def kernel(keys, table):
    M, G = keys.shape
    keys_grouped = keys.reshape(NS, NC, KEYS_PER_W).astype(jnp.int32)
    out = _sc_lookup(keys_grouped, table)
    return out.reshape(M, G, table.shape[1])


# speedup vs baseline: 1.0127x; 1.0127x over previous
import jax
import jax.numpy as jnp
from jax import lax
from jax.experimental import pallas as pl
from jax.experimental.pallas import tpu as pltpu
from jax.experimental.pallas import tpu_sc as plsc

TABLE_SIZE = 262144
MASK = TABLE_SIZE - 1
PRIMES = (6700417, 15485863, 32452843, 49979687)
NUM_HEADS = 4
D = 128
L = 16
NC, NS = 2, 16
N_KEYS = 4096 * 64
KEYS_PER_W = N_KEYS // (NC * NS)
CHUNK = 128
N_CHUNKS = KEYS_PER_W // CHUNK
NBUF = 4
N_QUADS = N_CHUNKS // NBUF


def _sc_body(keys_hbm, table_hbm, out_hbm, keys_v, idx_v, rows_v, sems):
    si = lax.axis_index("s")
    ci = lax.axis_index("c")
    gsem = [sems.at[jnp.int32(b)] for b in range(NBUF)]
    wsem = [sems.at[jnp.int32(NBUF + b)] for b in range(NBUF)]

    pltpu.sync_copy(keys_hbm.at[si, ci], keys_v)

    zvec = jnp.zeros((L,), jnp.float32)

    def _zero(buf):
        def _z(i, carry):
            for cc in range(D // L):
                rows_v[buf, i, pl.ds(cc * L, L)] = zvec
            return carry

        lax.fori_loop(jnp.int32(0), jnp.int32(CHUNK), _z, 0)

    def _hash_and_issue(g, buf):
        off = g * jnp.int32(CHUNK)
        for v in range(CHUNK // L):
            k = keys_v[pl.ds(off + jnp.int32(v * L), L)]
            sl = pl.ds(v * L, L)
            for j in range(NUM_HEADS):
                idx_v[buf, j, sl] = (k * jnp.int32(PRIMES[j])) & jnp.int32(MASK)
        for j in range(NUM_HEADS):
            for h in range(2):
                pltpu.async_copy(
                    table_hbm.at[
                        idx_v.at[jnp.int32(buf), jnp.int32(j), pl.ds(h * 64, 64)]
                    ],
                    rows_v.at[jnp.int32(buf), pl.ds(h * 64, 64)],
                    gsem[buf],
                    add=True,
                )

    def _wait_gathers(buf):
        for j in range(NUM_HEADS):
            for h in range(2):
                pltpu.make_async_copy(
                    table_hbm.at[
                        idx_v.at[jnp.int32(buf), jnp.int32(j), pl.ds(h * 64, 64)]
                    ],
                    rows_v.at[jnp.int32(buf), pl.ds(h * 64, 64)],
                    gsem[buf],
                ).wait()

    def _scale(buf):
        def _key(i, carry):
            for cc in range(D // L):
                sl = pl.ds(cc * L, L)
                rows_v[buf, i, sl] = rows_v[buf, i, sl] * jnp.float32(0.25)
            return carry

        lax.fori_loop(jnp.int32(0), jnp.int32(CHUNK), _key, 0)

    def _wait_write(buf):
        pltpu.make_async_copy(
            rows_v.at[jnp.int32(buf)], out_hbm.at[si, ci, jnp.int32(buf)], wsem[buf]
        ).wait()

    def _start_write(g, buf):
        pltpu.async_copy(
            rows_v.at[jnp.int32(buf)], out_hbm.at[si, ci, g], wsem[buf]
        )

    for b in range(NBUF):
        _zero(b)
    _hash_and_issue(jnp.int32(0), 0)
    _hash_and_issue(jnp.int32(1), 1)

    def _quad(q, carry):
        for k in range(NBUF):
            g = q * jnp.int32(NBUF) + jnp.int32(k)
            nb = (k + 2) % NBUF
            gn = g + jnp.int32(2)

            _wait_gathers(k)
            _scale(k)
            _start_write(g, k)

            @pl.when(jnp.logical_and(gn >= jnp.int32(NBUF), gn < jnp.int32(N_CHUNKS)))
            def _():
                _wait_write(nb)
                _zero(nb)

            @pl.when(gn < jnp.int32(N_CHUNKS))
            def _():
                _hash_and_issue(gn, nb)

        return carry

    lax.fori_loop(jnp.int32(0), jnp.int32(N_QUADS), _quad, 0)
    for b in range(NBUF):
        _wait_write(b)


@jax.jit
def _sc_lookup(keys_grouped, table):
    mesh = plsc.VectorSubcoreMesh(
        core_axis_name="c", subcore_axis_name="s", num_cores=NC, num_subcores=NS
    )
    f = pl.kernel(
        _sc_body,
        out_type=jax.ShapeDtypeStruct((NS, NC, N_CHUNKS, CHUNK, D), jnp.float32),
        mesh=mesh,
        scratch_types=[
            pltpu.VMEM((KEYS_PER_W,), jnp.int32),
            pltpu.VMEM((NBUF, NUM_HEADS, CHUNK), jnp.int32),
            pltpu.VMEM((NBUF, CHUNK, D), jnp.float32),
            pltpu.SemaphoreType.DMA((2 * NBUF,)),
        ],
    )
    return f(keys_grouped, table)


def kernel(keys, table):
    M, G = keys.shape
    keys_grouped = keys.reshape(NS, NC, KEYS_PER_W).astype(jnp.int32)
    out = _sc_lookup(keys_grouped, table)
    return out.reshape(M, G, table.shape[1])
